# baseline (speedup 1.0000x reference)
import functools

import jax
import jax.numpy as jnp
from jax import lax
from jax.experimental import pallas as pl
from jax.experimental.pallas import tpu as pltpu
from jax.experimental.pallas import tpu_sc as plsc

_NC = 2
_NS = 16
_NW = _NC * _NS
_C = 8
_LANES = 16
_NSLOT = 3


def kernel(x, token_table, pos_table):
    B, S = x.shape
    V, D = token_table.shape
    N = B * S
    ngrp = B // 2
    pos_per_w = S * ngrp // _NW
    nst = pos_per_w // _C

    xi = x.astype(jnp.int32)
    mesh = plsc.VectorSubcoreMesh(
        core_axis_name="c", subcore_axis_name="s",
        num_cores=_NC, num_subcores=_NS,
    )

    @functools.partial(
        pl.kernel,
        out_type=jax.ShapeDtypeStruct((N, D), jnp.float32),
        mesh=mesh,
        scratch_types=[
            pltpu.VMEM((2, pos_per_w), jnp.int32),
            [pltpu.VMEM((_C, D), jnp.float32) for _ in range(2 * _NSLOT)],
            pltpu.VMEM((_C, D), jnp.float32),
            [pltpu.SemaphoreType.DMA for _ in range(2 * _NSLOT)],
            [pltpu.SemaphoreType.DMA for _ in range(2 * _NSLOT)],
        ],
    )
    def k(x_hbm, tok_hbm, pos_hbm, out_hbm, idx_v, tb, pb, gsem, osem):
        wid = lax.axis_index("s") * _NC + lax.axis_index("c")
        g = lax.rem(wid, 2)
        b0 = 2 * g
        pos0 = (wid // 2) * pos_per_w

        def gather_pair(t, m):
            for e in range(2):
                pltpu.async_copy(
                    tok_hbm.at[idx_v.at[e, pl.ds(t * _C, _C)]],
                    tb[2 * m + e], gsem[2 * m + e])

        for e in range(2):
            pltpu.sync_copy(x_hbm.at[b0 + e, pl.ds(pos0, pos_per_w)],
                            idx_v.at[e])
        gather_pair(0, 0)
        gather_pair(1, 1)

        def step(t, m):
            pltpu.sync_copy(pos_hbm.at[pl.ds(pos0 + t * _C, _C)], pb)
            for e in range(2):
                pltpu.make_async_copy(
                    tok_hbm.at[idx_v.at[e, pl.ds(0, _C)]], tb[2 * m + e],
                    gsem[2 * m + e]).wait()

            def row(r, c2):
                for cb in range(D // _LANES):
                    sl = pl.ds(cb * _LANES, _LANES)
                    pv = pb[r, sl]
                    plsc.addupdate(tb[2 * m].at[r, sl], pv)
                    plsc.addupdate(tb[2 * m + 1].at[r, sl], pv)
                return c2

            lax.fori_loop(0, _C, row, 0)
            for e in range(2):
                pltpu.async_copy(
                    tb[2 * m + e],
                    out_hbm.at[pl.ds((b0 + e) * S + pos0 + t * _C, _C)],
                    osem[2 * m + e])

            mn = (m + 2) % _NSLOT

            @pl.when(t >= 1)
            def _drain_prev_out():
                for e in range(2):
                    pltpu.make_async_copy(
                        tb[2 * mn + e], out_hbm.at[pl.ds(0, _C)],
                        osem[2 * mn + e]).wait()

            @pl.when(t + 2 < nst)
            def _fire_next():
                gather_pair(t + 2, mn)

        def blk_body(blk, carry):
            for mm in range(_NSLOT):
                step(blk * _NSLOT + mm, mm)
            return carry

        lax.fori_loop(0, (nst - 1) // _NSLOT, blk_body, 0)
        step(nst - 1, (nst - 1) % _NSLOT)
        for e in range(2):
            pltpu.make_async_copy(
                tb[e], out_hbm.at[pl.ds(0, _C)],
                osem[e]).wait()

    out = k(xi, token_table, pos_table)
    return out.reshape(B, S, D)


# speedup vs baseline: 1.1111x; 1.1111x over previous
import functools

import jax
import jax.numpy as jnp
from jax import lax
from jax.experimental import pallas as pl
from jax.experimental.pallas import tpu as pltpu
from jax.experimental.pallas import tpu_sc as plsc

_NC = 2
_NS = 16
_NW = _NC * _NS
_P = 8
_H = 4
_LANES = 16
_NSLOT = 3


def kernel(x, token_table, pos_table):
    B, S = x.shape
    V, D = token_table.shape
    N = B * S
    ngrp = B // 2
    pos_per_w = S * ngrp // _NW
    nst = pos_per_w // _P
    nrows = 2 * _P

    x3 = x.astype(jnp.int32).reshape(B, S // _P, _P)
    mesh = plsc.VectorSubcoreMesh(
        core_axis_name="c", subcore_axis_name="s",
        num_cores=_NC, num_subcores=_NS,
    )

    @functools.partial(
        pl.kernel,
        out_type=jax.ShapeDtypeStruct((N, D), jnp.float32),
        mesh=mesh,
        scratch_types=[
            pltpu.VMEM((2, nst, _P), jnp.int32),
            [pltpu.VMEM((nrows, D), jnp.float32) for _ in range(_NSLOT)],
            [pltpu.VMEM((_H, D), jnp.float32) for _ in range(2)],
            [pltpu.SemaphoreType.DMA for _ in range(_NSLOT)],
            [pltpu.SemaphoreType.DMA for _ in range(_NSLOT)],
            [pltpu.SemaphoreType.DMA for _ in range(2)],
        ],
    )
    def k(x_hbm, tok_hbm, pos_hbm, out_hbm, idx_v, tb, pb, gsem, osem,
          psem):
        wid = lax.axis_index("s") * _NC + lax.axis_index("c")
        g = lax.rem(wid, 2)
        b0 = 2 * g
        band = wid // 2
        pos0 = band * pos_per_w

        for e in range(2):
            pltpu.sync_copy(
                x_hbm.at[b0 + e, pl.ds(band * nst, nst), :],
                idx_v.at[e])

        def gather(t, m):
            for e in range(2):
                pltpu.async_copy(
                    tok_hbm.at[idx_v.at[e, t]],
                    tb[m].at[pl.ds(e * _P, _P)], gsem[m])

        def pos_fetch(t, h):
            pltpu.async_copy(
                pos_hbm.at[pl.ds(pos0 + t * _P + h * _H, _H)], pb[h],
                psem[h])

        pos_fetch(0, 0)
        pos_fetch(0, 1)
        gather(0, 0)
        gather(1, 1)

        def half_add(m, h):
            def row(r, c2):
                for cb in range(D // _LANES):
                    sl = pl.ds(cb * _LANES, _LANES)
                    pv = pb[h][r, sl]
                    plsc.addupdate(tb[m].at[h * _H + r, sl], pv)
                    plsc.addupdate(tb[m].at[_P + h * _H + r, sl], pv)
                return c2

            lax.fori_loop(0, _H, row, 0)

        def step(t, m):
            for _ in range(2):
                pltpu.make_async_copy(
                    tok_hbm.at[idx_v.at[0, 0]],
                    tb[m].at[pl.ds(0, _P)], gsem[m]).wait()
            for h in range(2):
                pltpu.make_async_copy(
                    pos_hbm.at[pl.ds(0, _H)], pb[h], psem[h]).wait()
                half_add(m, h)

                @pl.when(t + 1 < nst)
                def _fire_next_pos():
                    pos_fetch(t + 1, h)

            for e in range(2):
                pltpu.async_copy(
                    tb[m].at[pl.ds(e * _P, _P)],
                    out_hbm.at[pl.ds((b0 + e) * S + pos0 + t * _P, _P)],
                    osem[m])

            mn = (m + 2) % _NSLOT

            @pl.when(t >= 1)
            def _drain_prev_out():
                for _ in range(2):
                    pltpu.make_async_copy(
                        tb[mn].at[pl.ds(0, _P)],
                        out_hbm.at[pl.ds(0, _P)], osem[mn]).wait()

            @pl.when(t + 2 < nst)
            def _fire_next():
                gather(t + 2, mn)

        def blk_body(blk, carry):
            for mm in range(_NSLOT):
                step(blk * _NSLOT + mm, mm)
            return carry

        lax.fori_loop(0, (nst - 1) // _NSLOT, blk_body, 0)
        step(nst - 1, (nst - 1) % _NSLOT)
        for _ in range(2):
            pltpu.make_async_copy(
                tb[(nst - 1) % _NSLOT].at[pl.ds(0, _P)],
                out_hbm.at[pl.ds(0, _P)],
                osem[(nst - 1) % _NSLOT]).wait()

    out = k(x3, token_table, pos_table)
    return out.reshape(B, S, D)


# speedup vs baseline: 1.2555x; 1.1300x over previous
import functools

import jax
import jax.numpy as jnp
from jax import lax
from jax.experimental import pallas as pl
from jax.experimental.pallas import tpu as pltpu
from jax.experimental.pallas import tpu_sc as plsc

_NC = 2
_NS = 16
_NW = _NC * _NS
_C = 8
_LANES = 16


def kernel(x, token_table, pos_table):
    B, S = x.shape
    V, D = token_table.shape
    N = B * S
    pos_per_w = S // _NW
    npc = pos_per_w // _C
    nchunks = npc * B

    xi = x.astype(jnp.int32)
    mesh = plsc.VectorSubcoreMesh(
        core_axis_name="c", subcore_axis_name="s",
        num_cores=_NC, num_subcores=_NS,
    )

    @functools.partial(
        pl.kernel,
        out_type=jax.ShapeDtypeStruct((N, D), jnp.float32),
        mesh=mesh,
        scratch_types=[
            pltpu.VMEM((B, pos_per_w), jnp.int32),
            [pltpu.VMEM((_C, D), jnp.float32) for _ in range(B)],
            [pltpu.VMEM((_C, D), jnp.float32) for _ in range(2)],
            [pltpu.SemaphoreType.DMA for _ in range(B)],
            [pltpu.SemaphoreType.DMA for _ in range(B)],
            [pltpu.SemaphoreType.DMA for _ in range(2)],
        ],
    )
    def k(x_hbm, tok_hbm, pos_hbm, out_hbm, idx_v, tb, pb, gsem, osem,
          psem):
        wid = lax.axis_index("s") * _NC + lax.axis_index("c")
        pos0 = wid * pos_per_w

        def gather(pcn, bn):
            pltpu.async_copy(
                tok_hbm.at[idx_v.at[bn, pl.ds(pcn * _C, _C)]],
                tb[bn], gsem[bn])

        def pos_fetch(pcn, u):
            pltpu.async_copy(
                pos_hbm.at[pl.ds(pos0 + pcn * _C, _C)], pb[u], psem[u])

        for b in range(B):
            pltpu.sync_copy(x_hbm.at[b, pl.ds(pos0, pos_per_w)],
                            idx_v.at[b])
        pos_fetch(0, 0)
        pos_fetch(1, 1)
        gather(0, 0)
        gather(0, 1)

        def pc2_body(pc2, carry):
            for u in range(2):
                pc = pc2 * 2 + u
                pltpu.make_async_copy(
                    pos_hbm.at[pl.ds(0, _C)], pb[u], psem[u]).wait()
                for b in range(B):
                    i = pc * B + b
                    pltpu.make_async_copy(
                        tok_hbm.at[idx_v.at[0, pl.ds(0, _C)]], tb[b],
                        gsem[b]).wait()

                    bn = (b + 2) % B
                    pcn = pc + (b + 2) // B

                    @pl.when(i >= 2)
                    def _drain_out_i2():
                        pltpu.make_async_copy(
                            tb[bn], out_hbm.at[pl.ds(0, _C)],
                            osem[bn]).wait()

                    @pl.when(pcn < npc)
                    def _fire_next():
                        gather(pcn, bn)

                    def row(r, c2):
                        for cb in range(D // _LANES):
                            sl = pl.ds(cb * _LANES, _LANES)
                            plsc.addupdate(tb[b].at[r, sl], pb[u][r, sl])
                        return c2

                    lax.fori_loop(0, _C, row, 0)
                    pltpu.async_copy(
                        tb[b],
                        out_hbm.at[pl.ds(b * S + pos0 + pc * _C, _C)],
                        osem[b])

                @pl.when(pc + 2 < npc)
                def _fire_next_pos():
                    pos_fetch(pc + 2, u)
            return carry

        lax.fori_loop(0, npc // 2, pc2_body, 0)
        for b in (2, 3):
            pltpu.make_async_copy(
                tb[b], out_hbm.at[pl.ds(0, _C)], osem[b]).wait()

    out = k(xi, token_table, pos_table)
    return out.reshape(B, S, D)


# speedup vs baseline: 1.4953x; 1.1910x over previous
import functools

import jax
import jax.numpy as jnp
from jax import lax
from jax.experimental import pallas as pl
from jax.experimental.pallas import tpu as pltpu
from jax.experimental.pallas import tpu_sc as plsc

_NC = 2
_NS = 16
_NW = _NC * _NS
_C = 8
_LANES = 16


def kernel(x, token_table, pos_table):
    B, S = x.shape
    V, D = token_table.shape
    N = B * S
    pos_per_w = S // _NW
    npc = pos_per_w // _C
    nchunks = npc * B

    xi = x.astype(jnp.int32)
    mesh = plsc.VectorSubcoreMesh(
        core_axis_name="c", subcore_axis_name="s",
        num_cores=_NC, num_subcores=_NS,
    )

    @functools.partial(
        pl.kernel,
        out_type=jax.ShapeDtypeStruct((N, D), jnp.float32),
        mesh=mesh,
        scratch_types=[
            pltpu.VMEM((B, pos_per_w), jnp.int32),
            pltpu.VMEM((_C,), jnp.int32),
            [pltpu.VMEM((_C, D), jnp.float32) for _ in range(B)],
            [pltpu.VMEM((_C, D), jnp.float32) for _ in range(2)],
            [pltpu.SemaphoreType.DMA for _ in range(B)],
            [pltpu.SemaphoreType.DMA for _ in range(B)],
            [pltpu.SemaphoreType.DMA for _ in range(2)],
        ],
    )
    def k(x_hbm, iota_hbm, tok_hbm, pos_hbm, out_hbm, idx_v, row_ids,
          tb, pb, gsem, osem, psem):
        wid = lax.axis_index("s") * _NC + lax.axis_index("c")
        pos0 = wid * pos_per_w

        def gather(pcn, bn):
            pltpu.async_copy(
                tok_hbm.at[idx_v.at[bn, pl.ds(pcn * _C, _C)]],
                tb[bn], gsem[bn])

        def pos_fetch(pcn, u):
            pltpu.async_copy(
                pos_hbm.at[pl.ds(pos0 + pcn * _C, _C)], pb[u], psem[u])

        for b in range(B):
            pltpu.sync_copy(x_hbm.at[b, pl.ds(pos0, pos_per_w)],
                            idx_v.at[b])
        pltpu.sync_copy(iota_hbm, row_ids)
        pos_fetch(0, 0)
        pos_fetch(1, 1)
        gather(0, 0)
        gather(0, 1)

        def pc2_body(pc2, carry):
            for u in range(2):
                pc = pc2 * 2 + u
                pltpu.make_async_copy(
                    pos_hbm.at[pl.ds(0, _C)], pb[u], psem[u]).wait()
                for b in range(B):
                    i = pc * B + b
                    pltpu.make_async_copy(
                        tok_hbm.at[idx_v.at[0, pl.ds(0, _C)]], tb[b],
                        gsem[b]).wait()

                    bn = (b + 2) % B
                    pcn = pc + (b + 2) // B

                    @pl.when(i >= 2)
                    def _drain_out_i2():
                        pltpu.make_async_copy(
                            tb[bn], out_hbm.at[pl.ds(0, _C)],
                            osem[bn]).wait()

                    @pl.when(pcn < npc)
                    def _fire_next():
                        gather(pcn, bn)

                    def row(r, c2):
                        @plsc.parallel_loop(0, D, _LANES, unroll=8)
                        def _cb(c):
                            sl = pl.ds(c, _LANES)
                            plsc.addupdate(tb[b].at[r, sl], pb[u][r, sl])
                        return c2

                    lax.fori_loop(0, _C, row, 0)
                    pltpu.async_copy(
                        tb[b],
                        out_hbm.at[pl.ds(b * S + pos0 + pc * _C, _C)],
                        osem[b])

                @pl.when(pc + 2 < npc)
                def _fire_next_pos():
                    pos_fetch(pc + 2, u)
            return carry

        lax.fori_loop(0, npc // 2, pc2_body, 0)
        for b in (2, 3):
            pltpu.make_async_copy(
                tb[b], out_hbm.at[pl.ds(0, _C)], osem[b]).wait()

    out = k(xi, jnp.arange(_C, dtype=jnp.int32), token_table, pos_table)
    return out.reshape(B, S, D)
